# baseline (speedup 1.0000x reference)
import functools

import jax
import jax.numpy as jnp
from jax import lax
from jax.experimental import pallas as pl
from jax.experimental.pallas import tpu as pltpu
from jax.experimental.pallas import tpu_sc as plsc

_TABLE = 40960
_H = 256
_B = 16384
_L = 20

_NC = 2
_NS = 16
_NW = _NC * _NS
_NSEG = 2 * _B
_SEG_PER_W = _NSEG // _NW
_CHUNK = 4
_NCHUNK = _SEG_PER_W // _CHUNK
_LANES = 16


def _sc_gather_sum(table, idx_flat):
    mesh = plsc.VectorSubcoreMesh(core_axis_name="c", subcore_axis_name="s")

    nbuf = 2
    ngrp = _NCHUNK // nbuf

    @functools.partial(
        pl.kernel,
        out_type=jax.ShapeDtypeStruct((_NSEG, _H), jnp.float32),
        mesh=mesh,
        scratch_types=[
            pltpu.VMEM((_SEG_PER_W * _L,), jnp.int32),
            [pltpu.VMEM((_CHUNK * _L, _H), jnp.float32) for _ in range(nbuf)],
            [pltpu.VMEM((_CHUNK, _H), jnp.float32) for _ in range(nbuf)],
            [pltpu.SemaphoreType.DMA for _ in range(nbuf)],
            [pltpu.SemaphoreType.DMA for _ in range(nbuf)],
        ],
    )
    def k(table_hbm, idx_hbm, out_hbm, idx_v, rows, outs, sems, osems):
        wid = lax.axis_index("s") * _NC + lax.axis_index("c")
        idx_base = wid * (_SEG_PER_W * _L)
        seg_base = wid * _SEG_PER_W
        pltpu.sync_copy(idx_hbm.at[pl.ds(idx_base, _SEG_PER_W * _L)], idx_v)

        def issue_gather(g, p):
            pltpu.async_copy(
                table_hbm.at[idx_v.at[pl.ds(g * (_CHUNK * _L), _CHUNK * _L)]],
                rows[p], sems[p])

        def wait_gather(p):
            pltpu.make_async_copy(table_hbm.at[pl.ds(0, _CHUNK * _L)],
                                  rows[p], sems[p]).wait()

        def accumulate(p):
            for c in range(_CHUNK):
                for hh in range(_H // _LANES):
                    sl = pl.ds(hh * _LANES, _LANES)
                    parts = []
                    for q in range(4):
                        s = rows[p][c * _L + 5 * q, sl]
                        for r in range(1, 5):
                            s = s + rows[p][c * _L + 5 * q + r, sl]
                        parts.append(s)
                    total = (parts[0] + parts[1]) + (parts[2] + parts[3])
                    outs[p][c, sl] = jnp.maximum(total, 0.0)

        def out_slice(g):
            return out_hbm.at[pl.ds(seg_base + g * _CHUNK, _CHUNK)]

        for p in range(nbuf):
            issue_gather(p, p)

        def body(j, _):
            for p in range(nbuf):
                g = j * nbuf + p
                wait_gather(p)

                @pl.when(j > 0)
                def _():
                    pltpu.make_async_copy(outs[p], out_slice(0),
                                          osems[p]).wait()

                accumulate(p)

                @pl.when(j < ngrp - 1)
                def _():
                    issue_gather(g + nbuf, p)

                pltpu.async_copy(outs[p], out_slice(g), osems[p])
            return ()

        lax.fori_loop(0, ngrp, body, (), unroll=False)
        for p in range(nbuf):
            pltpu.make_async_copy(outs[p], out_slice(0), osems[p]).wait()

    return k(table, idx_flat)


def _mlp_body(h_ref, w2_ref, b2_ref, w3_ref, b3_ref, w4_ref, b4_ref, out_ref):
    h = h_ref[...]
    z = jnp.maximum(
        jnp.dot(h, w2_ref[...], preferred_element_type=jnp.float32)
        + b2_ref[...], 0.0)
    z = jnp.maximum(
        jnp.dot(z, w3_ref[...], preferred_element_type=jnp.float32)
        + b3_ref[...], 0.0)
    out_ref[...] = jnp.sum(z * w4_ref[...], axis=1) + b4_ref[0, 0]


def _mlp(h, fc2_w, fc2_b, fc3_w, fc3_b, fc4_w, fc4_b):
    blk = 2048
    grid = (_B // blk,)
    full = lambda *s: pl.BlockSpec(s, lambda i: (0,) * len(s))
    return pl.pallas_call(
        _mlp_body,
        grid=grid,
        in_specs=[
            pl.BlockSpec((blk, 2 * _H), lambda i: (i, 0)),
            full(2 * _H, 32), full(1, 32),
            full(32, 32), full(1, 32),
            full(1, 32), full(1, 1),
        ],
        out_specs=pl.BlockSpec((blk,), lambda i: (i,)),
        out_shape=jax.ShapeDtypeStruct((_B,), jnp.float32),
    )(h, fc2_w.T, fc2_b.reshape(1, 32), fc3_w.T, fc3_b.reshape(1, 32),
      fc4_w.reshape(1, 32), fc4_b.reshape(1, 1))


def kernel(idx0_batch, idx1_batch, w1, fc2_w, fc2_b, fc3_w, fc3_b,
           fc4_w, fc4_b):
    table = w1.reshape(2 * _TABLE, _H)
    idx_all = jnp.stack([idx0_batch, idx1_batch + _TABLE], axis=1)
    idx_flat = idx_all.reshape(-1)
    h = _sc_gather_sum(table, idx_flat)
    h2 = h.reshape(_B, 2 * _H)
    return _mlp(h2, fc2_w, fc2_b, fc3_w, fc3_b, fc4_w, fc4_b)


# speedup vs baseline: 1.9312x; 1.9312x over previous
import functools

import jax
import jax.numpy as jnp
from jax import lax
from jax.experimental import pallas as pl
from jax.experimental.pallas import tpu as pltpu
from jax.experimental.pallas import tpu_sc as plsc

_TABLE = 40960
_H = 256
_B = 16384
_L = 20

_NC = 2
_NS = 16
_NW = _NC * _NS
_NSEG = 2 * _B
_SEG_PER_W = _NSEG // _NW
_CHUNK = 4
_NCHUNK = _SEG_PER_W // _CHUNK
_LANES = 16


def _sc_gather_sum(table, idx_flat):
    mesh = plsc.VectorSubcoreMesh(core_axis_name="c", subcore_axis_name="s")

    nbuf = 4
    ngrp = _NCHUNK // nbuf

    @functools.partial(
        pl.kernel,
        out_type=jax.ShapeDtypeStruct((_NSEG, _H), jnp.float32),
        mesh=mesh,
        scratch_types=[
            pltpu.VMEM((_SEG_PER_W * _L,), jnp.int32),
            [pltpu.VMEM((_CHUNK * _L, _H), jnp.float32) for _ in range(nbuf)],
            [pltpu.VMEM((_CHUNK, _H), jnp.float32) for _ in range(nbuf)],
            [pltpu.SemaphoreType.DMA for _ in range(nbuf)],
            [pltpu.SemaphoreType.DMA for _ in range(nbuf)],
        ],
    )
    def k(table_hbm, idx_hbm, out_hbm, idx_v, rows, outs, sems, osems):
        wid = lax.axis_index("s") * _NC + lax.axis_index("c")
        idx_base = wid * (_SEG_PER_W * _L)
        seg_base = wid * _SEG_PER_W
        pltpu.sync_copy(idx_hbm.at[pl.ds(idx_base, _SEG_PER_W * _L)], idx_v)

        def issue_gather(g, p):
            pltpu.async_copy(
                table_hbm.at[idx_v.at[pl.ds(g * (_CHUNK * _L), _CHUNK * _L)]],
                rows[p], sems[p])

        def wait_gather(p):
            pltpu.make_async_copy(table_hbm.at[pl.ds(0, _CHUNK * _L)],
                                  rows[p], sems[p]).wait()

        def accumulate(p):
            def seg_body(c, _):
                base = c * _L
                for hh in range(_H // _LANES):
                    sl = pl.ds(hh * _LANES, _LANES)
                    parts = []
                    for q in range(4):
                        s = rows[p][base + 5 * q, sl]
                        for r in range(1, 5):
                            s = s + rows[p][base + 5 * q + r, sl]
                        parts.append(s)
                    total = (parts[0] + parts[1]) + (parts[2] + parts[3])
                    outs[p][c, sl] = jnp.maximum(total, 0.0)
                return ()

            lax.fori_loop(0, _CHUNK, seg_body, (), unroll=False)

        def out_slice(g):
            return out_hbm.at[pl.ds(seg_base + g * _CHUNK, _CHUNK)]

        for p in range(nbuf):
            issue_gather(p, p)

        def body(j, _):
            for p in range(nbuf):
                g = j * nbuf + p
                wait_gather(p)

                @pl.when(j > 0)
                def _():
                    pltpu.make_async_copy(outs[p], out_slice(0),
                                          osems[p]).wait()

                accumulate(p)

                @pl.when(j < ngrp - 1)
                def _():
                    issue_gather(g + nbuf, p)

                pltpu.async_copy(outs[p], out_slice(g), osems[p])
            return ()

        lax.fori_loop(0, ngrp, body, (), unroll=False)
        for p in range(nbuf):
            pltpu.make_async_copy(outs[p], out_slice(0), osems[p]).wait()

    return k(table, idx_flat)


def _mlp_body(h_ref, w2_ref, b2_ref, w3_ref, b3_ref, w4_ref, b4_ref, out_ref):
    h = h_ref[...]
    z = jnp.maximum(
        jnp.dot(h, w2_ref[...], preferred_element_type=jnp.float32)
        + b2_ref[...], 0.0)
    z = jnp.maximum(
        jnp.dot(z, w3_ref[...], preferred_element_type=jnp.float32)
        + b3_ref[...], 0.0)
    out_ref[...] = jnp.sum(z * w4_ref[...], axis=1) + b4_ref[0, 0]


def _mlp(h, fc2_w, fc2_b, fc3_w, fc3_b, fc4_w, fc4_b):
    blk = 2048
    grid = (_B // blk,)
    full = lambda *s: pl.BlockSpec(s, lambda i: (0,) * len(s))
    return pl.pallas_call(
        _mlp_body,
        grid=grid,
        in_specs=[
            pl.BlockSpec((blk, 2 * _H), lambda i: (i, 0)),
            full(2 * _H, 32), full(1, 32),
            full(32, 32), full(1, 32),
            full(1, 32), full(1, 1),
        ],
        out_specs=pl.BlockSpec((blk,), lambda i: (i,)),
        out_shape=jax.ShapeDtypeStruct((_B,), jnp.float32),
    )(h, fc2_w.T, fc2_b.reshape(1, 32), fc3_w.T, fc3_b.reshape(1, 32),
      fc4_w.reshape(1, 32), fc4_b.reshape(1, 1))


def kernel(idx0_batch, idx1_batch, w1, fc2_w, fc2_b, fc3_w, fc3_b,
           fc4_w, fc4_b):
    table = w1.reshape(2 * _TABLE, _H)
    idx_all = jnp.stack([idx0_batch, idx1_batch + _TABLE], axis=1)
    idx_flat = idx_all.reshape(-1)
    h = _sc_gather_sum(table, idx_flat)
    h2 = h.reshape(_B, 2 * _H)
    return _mlp(h2, fc2_w, fc2_b, fc3_w, fc3_b, fc4_w, fc4_b)
